# Initial kernel scaffold; baseline (speedup 1.0000x reference)
import jax
import jax.numpy as jnp
import numpy as np
from jax import lax
from jax.experimental import pallas as pl
from jax.experimental.pallas import tpu as pltpu
from jax.experimental.pallas import tpu_sc as plsc

NH = 4
NB = 64
C = 16
D = 64
CH = 144
L = 9216
NCH = L // CH
BLK = 128
NBLK = L // BLK


def _a1_body(x_ref, w_ref, sxl_ref, sxr_ref, fw1_ref, fb1_ref, fw2_ref,
             fb2_ref, xe_ref, y_ref, mlp_ref):
    X = x_ref[0]
    Xf = X.reshape(64 * 96, 96)
    Xm = lax.dot_general(Xf, sxl_ref[...], (((1,), (0,)), ((), ()))).reshape(64, 96, 96)
    Xp = lax.dot_general(Xf, sxr_ref[...], (((1,), (0,)), ((), ()))).reshape(64, 96, 96)
    cols = (Xm, X, Xp)
    z = jnp.zeros((64, 1, 96), jnp.float32)
    acc = jnp.zeros((L, 144), jnp.float32)
    for ky in range(3):
        for kx in range(3):
            Xc = cols[kx]
            if ky == 0:
                Xt = jnp.concatenate([z, Xc[:, :95, :]], axis=1)
            elif ky == 2:
                Xt = jnp.concatenate([Xc[:, 1:, :], z], axis=1)
            else:
                Xt = Xc
            acc = acc + lax.dot_general(Xt.reshape(64, L), w_ref[ky, kx],
                                        (((0,), (1,)), ((), ())))
    fc = acc[:, 80:144]
    xe_ref[0] = acc[:, 0:16]
    y_ref[0] = acc[:, 16:80]
    h1 = jax.nn.relu(lax.dot_general(fc, fw1_ref[...], (((1,), (1,)), ((), ())))
                     + fb1_ref[...][None, :])
    mlp_ref[0] = lax.dot_general(h1, fw2_ref[...], (((1,), (1,)), ((), ()))) \
        + fb2_ref[...][None, :]


def _a2_body(xe_ref, rot_ref, tinc_ref, texc_ref, slb_ref, g_ref, pos_ref,
             scr_ref):
    xe = xe_ref[0]
    ohs = []
    for h in range(NH):
        r = lax.dot_general(xe, rot_ref[...][:, h * NB:(h + 1) * NB],
                            (((1,), (0,)), ((), ())))
        m = jnp.max(r, axis=1, keepdims=True)
        io = lax.broadcasted_iota(jnp.float32, (L, NB), 1)
        code = jnp.min(jnp.where(r == m, io, 1e9), axis=1, keepdims=True)
        ohs.append((io == code).astype(jnp.float32))
    O = jnp.concatenate(ohs, axis=1)
    bsums = []
    for b in range(NBLK):
        osl = lax.slice(O, (b * BLK, 0), (b * BLK + BLK, NH * NB))
        incl = lax.dot_general(tinc_ref[...], osl, (((1,), (0,)), ((), ())))
        scr_ref[pl.ds(b * BLK, BLK), :] = incl
        bsums.append(jnp.sum(osl, axis=0, keepdims=True))
    bs = jnp.concatenate(bsums, axis=0)
    prev = lax.dot_general(texc_ref[...], bs, (((1,), (0,)), ((), ())))
    hist = jnp.sum(bs, axis=0, keepdims=True)
    excl = lax.dot_general(hist, slb_ref[...], (((1,), (0,)), ((), ())))
    for b in range(NBLK):
        osl = lax.slice(O, (b * BLK, 0), (b * BLK + BLK, NH * NB))
        val = scr_ref[pl.ds(b * BLK, BLK), :] - 1.0 \
            + prev[b:b + 1, :] + excl
        posb = lax.dot_general(osl * val, g_ref[...], (((1,), (0,)), ((), ())))
        pos_ref[0, pl.ds(b * BLK, BLK), :] = posb.astype(jnp.int32)


def _b_body(gpos_hbm, x_hbm, y_hbm, mlp_hbm, xs_hbm, ys_hbm, ms_hbm,
            idx_v, bx, by, bm, sem):
    wid = lax.axis_index("s") * 2 + lax.axis_index("c")
    bh = wid // 4
    q = wid % 4
    b = bh // NH
    t0 = q * (L // 4)
    nchunks = (L // 4) // BLK
    for j in range(nchunks):
        goff = bh * L + t0 + j * BLK
        soff = b * L + t0 + j * BLK
        pltpu.sync_copy(gpos_hbm.at[pl.ds(goff, BLK)], idx_v.at[j])
        pltpu.sync_copy(x_hbm.at[pl.ds(soff, BLK)], bx)
        pltpu.async_copy(bx, xs_hbm.at[idx_v.at[j]], sem).wait()
        pltpu.sync_copy(y_hbm.at[pl.ds(soff, BLK)], by)
        pltpu.async_copy(by, ys_hbm.at[idx_v.at[j]], sem).wait()
        pltpu.sync_copy(mlp_hbm.at[pl.ds(soff, BLK)], bm)
        pltpu.async_copy(bm, ms_hbm.at[idx_v.at[j]], sem).wait()


def _c_body(xc_ref, xp_ref, xn_ref, yc_ref, yp_ref, yn_ref, mc_ref, mp_ref,
            mn_ref, ret_ref, bsc_ref):
    xc = xc_ref[0, 0]
    xh = jnp.concatenate([xc, xp_ref[0, 0], xn_ref[0, 0]], axis=0)
    nrm = jnp.sqrt(jnp.sum(xh * xh, axis=1, keepdims=True))
    xmn = xh / jnp.maximum(nrm, 5e-05)
    mh = jnp.concatenate([mc_ref[0, 0], mp_ref[0, 0], mn_ref[0, 0]], axis=0)
    rawT = lax.dot_general(xmn, xc, (((1,), (1,)), ((), ()))) + mh
    mx = jnp.max(rawT, axis=0, keepdims=True)
    ex = jnp.exp(rawT - mx)
    s = jnp.sum(ex, axis=0, keepdims=True)
    bsc_ref[0, 0, 0] = jnp.log(s) + mx
    scT = ex / s
    yh = jnp.concatenate([yc_ref[0, 0], yp_ref[0, 0], yn_ref[0, 0]], axis=0)
    ret_ref[0, 0] = lax.dot_general(scT, yh, (((0,), (0,)), ((), ())))


def _d_body(gpos_hbm, rets_hbm, bscs_hbm, retu_hbm, bscu_hbm,
            idx_v, buf, scv, obuf, sem):
    wid = lax.axis_index("s") * 2 + lax.axis_index("c")
    bh = wid // 4
    q = wid % 4
    t0 = q * (L // 4)
    nchunks = (L // 4) // BLK
    pltpu.sync_copy(bscs_hbm.at[pl.ds(bh * L, L)], scv)
    base = jnp.full((16,), bh * L, jnp.int32)
    for j in range(nchunks):
        goff = bh * L + t0 + j * BLK
        pltpu.sync_copy(gpos_hbm.at[pl.ds(goff, BLK)], idx_v.at[j])
        pltpu.async_copy(rets_hbm.at[idx_v.at[j]], buf, sem).wait()
        pltpu.sync_copy(buf, retu_hbm.at[pl.ds(goff, BLK)])
        for g in range(BLK // 16):
            iv = idx_v[j, pl.ds(g * 16, 16)] - base
            obuf[pl.ds(j * BLK + g * 16, 16)] = plsc.load_gather(scv, [iv])
    pltpu.sync_copy(obuf, bscu_hbm.at[pl.ds(bh * L + t0, L // 4)])


def _e_body(ret_ref, bsc_ref, inp_ref, it_ref, out_ref):
    bsc = bsc_ref[0]
    i4 = jnp.eye(NH, dtype=jnp.float32)
    bscT = lax.dot_general(bsc, i4, (((0,), (0,)), ((), ())))
    mx = jnp.max(bscT, axis=1, keepdims=True)
    e = jnp.exp(bscT - mx)
    p = e / jnp.sum(e, axis=1, keepdims=True)
    acc = jnp.zeros((512, D), jnp.float32)
    for h in range(NH):
        acc = acc + ret_ref[0, h] * p[:, h:h + 1]
    outc = lax.dot_general(acc, it_ref[...], (((0,), (0,)), ((), ())))
    out_ref[0] = outc + inp_ref[0]


def _sc_scatter(gpos, x_flat, y_flat, mlp_flat):
    mesh = plsc.VectorSubcoreMesh(core_axis_name="c", subcore_axis_name="s")
    M = 2 * NH * L
    nchunks = (L // 4) // BLK
    f = pl.kernel(
        _b_body,
        out_type=(
            jax.ShapeDtypeStruct((M, C), jnp.float32),
            jax.ShapeDtypeStruct((M, D), jnp.float32),
            jax.ShapeDtypeStruct((M, CH), jnp.float32),
        ),
        mesh=mesh,
        scratch_types=[
            pltpu.VMEM((nchunks, BLK), jnp.int32),
            pltpu.VMEM((BLK, C), jnp.float32),
            pltpu.VMEM((BLK, D), jnp.float32),
            pltpu.VMEM((BLK, CH), jnp.float32),
            pltpu.SemaphoreType.DMA,
        ],
    )
    return f(gpos, x_flat, y_flat, mlp_flat)


def _sc_gather(gpos, ret_s, bsc_s):
    mesh = plsc.VectorSubcoreMesh(core_axis_name="c", subcore_axis_name="s")
    M = 2 * NH * L
    nchunks = (L // 4) // BLK
    f = pl.kernel(
        _d_body,
        out_type=(
            jax.ShapeDtypeStruct((M, D), jnp.float32),
            jax.ShapeDtypeStruct((M,), jnp.float32),
        ),
        mesh=mesh,
        scratch_types=[
            pltpu.VMEM((nchunks, BLK), jnp.int32),
            pltpu.VMEM((BLK, D), jnp.float32),
            pltpu.VMEM((L,), jnp.float32),
            pltpu.VMEM((L // 4,), jnp.float32),
            pltpu.SemaphoreType.DMA,
        ],
    )
    return f(gpos, ret_s, bsc_s)


def kernel(input, w_match, w_assembly, w_assembly_fc, fc_w1, fc_b1, fc_w2,
           fc_b2, rotations):
    N = input.shape[0]
    Wc = jnp.concatenate([w_match, w_assembly, w_assembly_fc], axis=0)
    Wc = Wc.transpose(2, 3, 0, 1)
    sxl = jnp.asarray(np.eye(96, k=1), jnp.float32)
    sxr = jnp.asarray(np.eye(96, k=-1), jnp.float32)
    rot2 = rotations.reshape(C, NH * NB)

    xe, y, mlp = pl.pallas_call(
        _a1_body,
        grid=(N,),
        in_specs=[
            pl.BlockSpec((1, 64, 96, 96), lambda n: (n, 0, 0, 0)),
            pl.BlockSpec((3, 3, 144, 64), lambda n: (0, 0, 0, 0)),
            pl.BlockSpec((96, 96), lambda n: (0, 0)),
            pl.BlockSpec((96, 96), lambda n: (0, 0)),
            pl.BlockSpec((144, 64), lambda n: (0, 0)),
            pl.BlockSpec((144,), lambda n: (0,)),
            pl.BlockSpec((144, 144), lambda n: (0, 0)),
            pl.BlockSpec((144,), lambda n: (0,)),
        ],
        out_specs=[
            pl.BlockSpec((1, L, C), lambda n: (n, 0, 0)),
            pl.BlockSpec((1, L, D), lambda n: (n, 0, 0)),
            pl.BlockSpec((1, L, CH), lambda n: (n, 0, 0)),
        ],
        out_shape=[
            jax.ShapeDtypeStruct((N, L, C), jnp.float32),
            jax.ShapeDtypeStruct((N, L, D), jnp.float32),
            jax.ShapeDtypeStruct((N, L, CH), jnp.float32),
        ],
    )(input, Wc, sxl, sxr, fc_w1, fc_b1, fc_w2, fc_b2)

    tinc = jnp.asarray(np.tril(np.ones((BLK, BLK))), jnp.float32)
    texc = jnp.asarray(np.tril(np.ones((NBLK, NBLK)), k=-1), jnp.float32)
    dd = np.arange(NB)
    slb_small = (dd[:, None] < dd[None, :]).astype(np.float32)
    slb = jnp.asarray(np.kron(np.eye(NH), slb_small), jnp.float32)
    gmat = jnp.asarray(np.kron(np.eye(NH), np.ones((NB, 1))), jnp.float32)

    pos = pl.pallas_call(
        _a2_body,
        grid=(N,),
        in_specs=[
            pl.BlockSpec((1, L, C), lambda n: (n, 0, 0)),
            pl.BlockSpec((C, NH * NB), lambda n: (0, 0)),
            pl.BlockSpec((BLK, BLK), lambda n: (0, 0)),
            pl.BlockSpec((NBLK, NBLK), lambda n: (0, 0)),
            pl.BlockSpec((NH * NB, NH * NB), lambda n: (0, 0)),
            pl.BlockSpec((NH * NB, NH), lambda n: (0, 0)),
        ],
        out_specs=pl.BlockSpec((1, L, NH), lambda n: (n, 0, 0)),
        out_shape=jax.ShapeDtypeStruct((N, L, NH), jnp.int32),
        scratch_shapes=[pltpu.VMEM((L, NH * NB), jnp.float32)],
    )(xe, rot2, tinc, texc, slb, gmat)

    offs = (jnp.arange(N, dtype=jnp.int32)[:, None, None] * NH
            + jnp.arange(NH, dtype=jnp.int32)[None, :, None]) * L
    gpos = (pos.transpose(0, 2, 1) + offs).reshape(-1)

    xs, ys, ms = _sc_scatter(gpos, xe.reshape(N * L, C), y.reshape(N * L, D),
                             mlp.reshape(N * L, CH))

    xs4 = xs.reshape(N, NH, L, C)
    ys4 = ys.reshape(N, NH, L, D)
    ms4 = ms.reshape(N, NH, L, CH)

    def ctr(n, h, k):
        return (n, h, k, 0)

    def prv(n, h, k):
        return (n, h, (k - 1) % NCH, 0)

    def nxt(n, h, k):
        return (n, h, (k + 1) % NCH, 0)

    def mk(dim, imap):
        return pl.BlockSpec((1, 1, CH, dim), imap)

    ret_s, bsc_s = pl.pallas_call(
        _c_body,
        grid=(N, NH, NCH),
        in_specs=[mk(C, ctr), mk(C, prv), mk(C, nxt),
                  mk(D, ctr), mk(D, prv), mk(D, nxt),
                  mk(CH, ctr), mk(CH, prv), mk(CH, nxt)],
        out_specs=[
            pl.BlockSpec((1, 1, CH, D),
                         lambda n, h, k: (n, h, k, 0)),
            pl.BlockSpec((1, 1, 1, 1, CH),
                         lambda n, h, k: (n, h, k, 0, 0)),
        ],
        out_shape=[
            jax.ShapeDtypeStruct((N, NH, L, D), jnp.float32),
            jax.ShapeDtypeStruct((N, NH, NCH, 1, CH), jnp.float32),
        ],
    )(xs4, xs4, xs4, ys4, ys4, ys4, ms4, ms4, ms4)

    ret_u, bsc_u = _sc_gather(gpos, ret_s.reshape(N * NH * L, D),
                              bsc_s.reshape(N * NH * L))

    ret_u = ret_u.reshape(N, NH, L, D)
    bsc_u = bsc_u.reshape(N, NH, L)
    it512 = jnp.asarray(np.eye(512), jnp.float32)
    inp3 = input.reshape(N, 64, L)

    TB = 512
    out = pl.pallas_call(
        _e_body,
        grid=(N, L // TB),
        in_specs=[
            pl.BlockSpec((1, NH, TB, D), lambda n, t: (n, 0, t, 0)),
            pl.BlockSpec((1, NH, TB), lambda n, t: (n, 0, t)),
            pl.BlockSpec((1, 64, TB), lambda n, t: (n, 0, t)),
            pl.BlockSpec((TB, TB), lambda n, t: (0, 0)),
        ],
        out_specs=pl.BlockSpec((1, 64, TB), lambda n, t: (n, 0, t)),
        out_shape=jax.ShapeDtypeStruct((N, 64, L), jnp.float32),
    )(ret_u, bsc_u, inp3, it512)

    return out.reshape(input.shape)


# speedup vs baseline: 135.2466x; 135.2466x over previous
import jax
import jax.numpy as jnp
import numpy as np
from jax import lax
from jax.experimental import pallas as pl
from jax.experimental.pallas import tpu as pltpu
from jax.experimental.pallas import tpu_sc as plsc

NH = 4
NB = 64
C = 16
D = 64
CH = 144
L = 9216
NCH = L // CH
BLK = 128
NBLK = L // BLK


A1B = 1536
PAD = 128


def _a1_body(x_ref, w_ref, fw1_ref, fb1_ref, fw2_ref,
             fb2_ref, xe_ref, y_ref, mlp_ref):
    for j in range(L // A1B):
        col = lax.rem(lax.broadcasted_iota(jnp.int32, (64, A1B), 1)
                      + (j * A1B), 96)
        mask_first = (col != 0).astype(jnp.float32)
        mask_last = (col != 95).astype(jnp.float32)
        acc = jnp.zeros((A1B, 144), jnp.float32)
        for ky in range(3):
            for kx in range(3):
                sh = (ky - 1) * 96 + (kx - 1)
                Xt = x_ref[0, :, pl.ds(PAD + j * A1B + sh, A1B)]
                if kx == 0:
                    Xt = Xt * mask_first
                elif kx == 2:
                    Xt = Xt * mask_last
                acc = acc + lax.dot_general(Xt, w_ref[ky, kx],
                                            (((0,), (1,)), ((), ())))
        fc = acc[:, 80:144]
        xe_ref[0, pl.ds(j * A1B, A1B), :] = acc[:, 0:16]
        y_ref[0, pl.ds(j * A1B, A1B), :] = acc[:, 16:80]
        h1 = jax.nn.relu(
            lax.dot_general(fc, fw1_ref[...], (((1,), (1,)), ((), ())))
            + fb1_ref[...][None, :])
        mlp_ref[0, pl.ds(j * A1B, A1B), :] = \
            lax.dot_general(h1, fw2_ref[...], (((1,), (1,)), ((), ()))) \
            + fb2_ref[...][None, :]


def _a2_body(xe_ref, rot_ref, tinc_ref, texc_ref, slb_ref, g_ref, pos_ref,
             scr_ref):
    xe = xe_ref[0]
    ohs = []
    for h in range(NH):
        r = lax.dot_general(xe, rot_ref[...][:, h * NB:(h + 1) * NB],
                            (((1,), (0,)), ((), ())))
        m = jnp.max(r, axis=1, keepdims=True)
        io = lax.broadcasted_iota(jnp.int32, (L, NB), 1).astype(jnp.float32)
        code = jnp.min(jnp.where(r == m, io, 1e9), axis=1, keepdims=True)
        ohs.append((io == code).astype(jnp.float32))
    O = jnp.concatenate(ohs, axis=1)
    bsums = []
    for b in range(NBLK):
        osl = lax.slice(O, (b * BLK, 0), (b * BLK + BLK, NH * NB))
        incl = lax.dot_general(tinc_ref[...], osl, (((1,), (0,)), ((), ())))
        scr_ref[pl.ds(b * BLK, BLK), :] = incl
        bsums.append(jnp.sum(osl, axis=0, keepdims=True))
    bs = jnp.concatenate(bsums, axis=0)
    prev = lax.dot_general(texc_ref[...], bs, (((1,), (0,)), ((), ())))
    hist = jnp.sum(bs, axis=0, keepdims=True)
    excl = lax.dot_general(hist, slb_ref[...], (((1,), (0,)), ((), ())))
    for b in range(NBLK):
        osl = lax.slice(O, (b * BLK, 0), (b * BLK + BLK, NH * NB))
        val = scr_ref[pl.ds(b * BLK, BLK), :] - 1.0 \
            + prev[b:b + 1, :] + excl
        posb = lax.dot_general(osl * val, g_ref[...], (((1,), (0,)), ((), ())))
        pos_ref[0, pl.ds(b * BLK, BLK), :] = posb.astype(jnp.int32)


def _b_body(gpos_hbm, x_hbm, y_hbm, mlp_hbm, xs_hbm, ys_hbm, ms_hbm,
            idx_v, bx, by, bm, sem):
    wid = lax.axis_index("s") * 2 + lax.axis_index("c")
    bh = wid // 4
    q = wid % 4
    b = bh // NH
    t0 = q * (L // 4)
    nchunks = (L // 4) // BLK
    for j in range(nchunks):
        goff = bh * L + t0 + j * BLK
        soff = b * L + t0 + j * BLK
        pltpu.sync_copy(gpos_hbm.at[pl.ds(goff, BLK)], idx_v.at[j])
        pltpu.sync_copy(x_hbm.at[pl.ds(soff, BLK)], bx)
        pltpu.async_copy(bx, xs_hbm.at[idx_v.at[j]], sem).wait()
        pltpu.sync_copy(y_hbm.at[pl.ds(soff, BLK)], by)
        pltpu.async_copy(by, ys_hbm.at[idx_v.at[j]], sem).wait()
        pltpu.sync_copy(mlp_hbm.at[pl.ds(soff, BLK)], bm)
        pltpu.async_copy(bm, ms_hbm.at[idx_v.at[j]], sem).wait()


def _c_body(xc_ref, xp_ref, xn_ref, yc_ref, yp_ref, yn_ref, mc_ref, mp_ref,
            mn_ref, ret_ref, bsc_ref):
    xc = xc_ref[0, 0]
    xh = jnp.concatenate([xc, xp_ref[0, 0], xn_ref[0, 0]], axis=0)
    nrm = jnp.sqrt(jnp.sum(xh * xh, axis=1, keepdims=True))
    xmn = xh / jnp.maximum(nrm, 5e-05)
    mh = jnp.concatenate([mc_ref[0, 0], mp_ref[0, 0], mn_ref[0, 0]], axis=0)
    rawT = lax.dot_general(xmn, xc, (((1,), (1,)), ((), ()))) + mh
    mx = jnp.max(rawT, axis=0, keepdims=True)
    ex = jnp.exp(rawT - mx)
    s = jnp.sum(ex, axis=0, keepdims=True)
    bsc_ref[0, 0, 0] = jnp.log(s) + mx
    scT = ex / s
    yh = jnp.concatenate([yc_ref[0, 0], yp_ref[0, 0], yn_ref[0, 0]], axis=0)
    ret_ref[0, 0] = lax.dot_general(scT, yh, (((0,), (0,)), ((), ())))


def _d_body(gpos_hbm, rets_hbm, bscs_hbm, retu_hbm, bscu_hbm,
            idx_v, buf, scv, obuf, sem):
    wid = lax.axis_index("s") * 2 + lax.axis_index("c")
    bh = wid // 4
    q = wid % 4
    t0 = q * (L // 4)
    nchunks = (L // 4) // BLK
    pltpu.sync_copy(bscs_hbm.at[pl.ds(bh * L, L)], scv)
    base = jnp.full((16,), bh * L, jnp.int32)
    for j in range(nchunks):
        goff = bh * L + t0 + j * BLK
        pltpu.sync_copy(gpos_hbm.at[pl.ds(goff, BLK)], idx_v.at[j])
        pltpu.async_copy(rets_hbm.at[idx_v.at[j]], buf, sem).wait()
        pltpu.sync_copy(buf, retu_hbm.at[pl.ds(goff, BLK)])
        for g in range(BLK // 16):
            iv = idx_v[j, pl.ds(g * 16, 16)] - base
            obuf[pl.ds(j * BLK + g * 16, 16)] = plsc.load_gather(scv, [iv])
    pltpu.sync_copy(obuf, bscu_hbm.at[pl.ds(bh * L + t0, L // 4)])


def _e_body(ret_ref, bsc_ref, inp_ref, it_ref, out_ref):
    bsc = bsc_ref[0]
    i4 = jnp.eye(NH, dtype=jnp.float32)
    bscT = lax.dot_general(bsc, i4, (((0,), (0,)), ((), ())))
    mx = jnp.max(bscT, axis=1, keepdims=True)
    e = jnp.exp(bscT - mx)
    p = e / jnp.sum(e, axis=1, keepdims=True)
    acc = jnp.zeros((512, D), jnp.float32)
    for h in range(NH):
        acc = acc + ret_ref[0, h] * p[:, h:h + 1]
    outc = lax.dot_general(acc, it_ref[...], (((0,), (0,)), ((), ())))
    out_ref[0] = outc + inp_ref[0]


def _sc_scatter(gpos, x_flat, y_flat, mlp_flat):
    mesh = plsc.VectorSubcoreMesh(core_axis_name="c", subcore_axis_name="s")
    M = 2 * NH * L
    nchunks = (L // 4) // BLK
    f = pl.kernel(
        _b_body,
        out_type=(
            jax.ShapeDtypeStruct((M, C), jnp.float32),
            jax.ShapeDtypeStruct((M, D), jnp.float32),
            jax.ShapeDtypeStruct((M, CH), jnp.float32),
        ),
        mesh=mesh,
        scratch_types=[
            pltpu.VMEM((nchunks, BLK), jnp.int32),
            pltpu.VMEM((BLK, C), jnp.float32),
            pltpu.VMEM((BLK, D), jnp.float32),
            pltpu.VMEM((BLK, CH), jnp.float32),
            pltpu.SemaphoreType.DMA,
        ],
        compiler_params=pltpu.CompilerParams(use_tc_tiling_on_sc=False, needs_layout_passes=False),
    )
    return f(gpos, x_flat, y_flat, mlp_flat)


def _sc_gather(gpos, ret_s, bsc_s):
    mesh = plsc.VectorSubcoreMesh(core_axis_name="c", subcore_axis_name="s")
    M = 2 * NH * L
    nchunks = (L // 4) // BLK
    f = pl.kernel(
        _d_body,
        out_type=(
            jax.ShapeDtypeStruct((M, D), jnp.float32),
            jax.ShapeDtypeStruct((M,), jnp.float32),
        ),
        mesh=mesh,
        scratch_types=[
            pltpu.VMEM((nchunks, BLK), jnp.int32),
            pltpu.VMEM((BLK, D), jnp.float32),
            pltpu.VMEM((L,), jnp.float32),
            pltpu.VMEM((L // 4,), jnp.float32),
            pltpu.SemaphoreType.DMA,
        ],
        compiler_params=pltpu.CompilerParams(use_tc_tiling_on_sc=False, needs_layout_passes=False),
    )
    return f(gpos, ret_s, bsc_s)


def kernel(input, w_match, w_assembly, w_assembly_fc, fc_w1, fc_b1, fc_w2,
           fc_b2, rotations):
    N = input.shape[0]
    Wc = jnp.concatenate([w_match, w_assembly, w_assembly_fc], axis=0)
    Wc = Wc.transpose(2, 3, 0, 1)
    rot2 = rotations.reshape(C, NH * NB)
    inp3 = input.reshape(N, 64, L)
    inp_pad = jnp.pad(inp3, ((0, 0), (0, 0), (PAD, PAD)))

    xe, y, mlp = pl.pallas_call(
        _a1_body,
        grid=(N,),
        in_specs=[
            pl.BlockSpec((1, 64, L + 2 * PAD), lambda n: (n, 0, 0)),
            pl.BlockSpec((3, 3, 144, 64), lambda n: (0, 0, 0, 0)),
            pl.BlockSpec((144, 64), lambda n: (0, 0)),
            pl.BlockSpec((144,), lambda n: (0,)),
            pl.BlockSpec((144, 144), lambda n: (0, 0)),
            pl.BlockSpec((144,), lambda n: (0,)),
        ],
        out_specs=[
            pl.BlockSpec((1, L, C), lambda n: (n, 0, 0)),
            pl.BlockSpec((1, L, D), lambda n: (n, 0, 0)),
            pl.BlockSpec((1, L, CH), lambda n: (n, 0, 0)),
        ],
        out_shape=[
            jax.ShapeDtypeStruct((N, L, C), jnp.float32),
            jax.ShapeDtypeStruct((N, L, D), jnp.float32),
            jax.ShapeDtypeStruct((N, L, CH), jnp.float32),
        ],
    )(inp_pad, Wc, fc_w1, fc_b1, fc_w2, fc_b2)

    tinc = jnp.asarray(np.tril(np.ones((BLK, BLK))), jnp.float32)
    texc = jnp.asarray(np.tril(np.ones((NBLK, NBLK)), k=-1), jnp.float32)
    dd = np.arange(NB)
    slb_small = (dd[:, None] < dd[None, :]).astype(np.float32)
    slb = jnp.asarray(np.kron(np.eye(NH), slb_small), jnp.float32)
    gmat = jnp.asarray(np.kron(np.eye(NH), np.ones((NB, 1))), jnp.float32)

    pos = pl.pallas_call(
        _a2_body,
        grid=(N,),
        in_specs=[
            pl.BlockSpec((1, L, C), lambda n: (n, 0, 0)),
            pl.BlockSpec((C, NH * NB), lambda n: (0, 0)),
            pl.BlockSpec((BLK, BLK), lambda n: (0, 0)),
            pl.BlockSpec((NBLK, NBLK), lambda n: (0, 0)),
            pl.BlockSpec((NH * NB, NH * NB), lambda n: (0, 0)),
            pl.BlockSpec((NH * NB, NH), lambda n: (0, 0)),
        ],
        out_specs=pl.BlockSpec((1, L, NH), lambda n: (n, 0, 0)),
        out_shape=jax.ShapeDtypeStruct((N, L, NH), jnp.int32),
        scratch_shapes=[pltpu.VMEM((L, NH * NB), jnp.float32)],
    )(xe, rot2, tinc, texc, slb, gmat)

    offs = (jnp.arange(N, dtype=jnp.int32)[:, None, None] * NH
            + jnp.arange(NH, dtype=jnp.int32)[None, :, None]) * L
    gpos = (pos.transpose(0, 2, 1) + offs).reshape(-1)

    xs, ys, ms = _sc_scatter(gpos, xe.reshape(N * L, C), y.reshape(N * L, D),
                             mlp.reshape(N * L, CH))

    xs4 = xs.reshape(N, NH, L, C)
    ys4 = ys.reshape(N, NH, L, D)
    ms4 = ms.reshape(N, NH, L, CH)

    def ctr(n, h, k):
        return (n, h, k, 0)

    def prv(n, h, k):
        return (n, h, (k - 1) % NCH, 0)

    def nxt(n, h, k):
        return (n, h, (k + 1) % NCH, 0)

    def mk(dim, imap):
        return pl.BlockSpec((1, 1, CH, dim), imap)

    ret_s, bsc_s = pl.pallas_call(
        _c_body,
        grid=(N, NH, NCH),
        in_specs=[mk(C, ctr), mk(C, prv), mk(C, nxt),
                  mk(D, ctr), mk(D, prv), mk(D, nxt),
                  mk(CH, ctr), mk(CH, prv), mk(CH, nxt)],
        out_specs=[
            pl.BlockSpec((1, 1, CH, D),
                         lambda n, h, k: (n, h, k, 0)),
            pl.BlockSpec((1, 1, 1, 1, CH),
                         lambda n, h, k: (n, h, k, 0, 0)),
        ],
        out_shape=[
            jax.ShapeDtypeStruct((N, NH, L, D), jnp.float32),
            jax.ShapeDtypeStruct((N, NH, NCH, 1, CH), jnp.float32),
        ],
    )(xs4, xs4, xs4, ys4, ys4, ys4, ms4, ms4, ms4)

    ret_u, bsc_u = _sc_gather(gpos, ret_s.reshape(N * NH * L, D),
                              bsc_s.reshape(N * NH * L))

    ret_u = ret_u.reshape(N, NH, L, D)
    bsc_u = bsc_u.reshape(N, NH, L)
    it512 = jnp.asarray(np.eye(512), jnp.float32)
    inp3 = input.reshape(N, 64, L)

    TB = 512
    out = pl.pallas_call(
        _e_body,
        grid=(N, L // TB),
        in_specs=[
            pl.BlockSpec((1, NH, TB, D), lambda n, t: (n, 0, t, 0)),
            pl.BlockSpec((1, NH, TB), lambda n, t: (n, 0, t)),
            pl.BlockSpec((1, 64, TB), lambda n, t: (n, 0, t)),
            pl.BlockSpec((TB, TB), lambda n, t: (0, 0)),
        ],
        out_specs=pl.BlockSpec((1, 64, TB), lambda n, t: (n, 0, t)),
        out_shape=jax.ShapeDtypeStruct((N, 64, L), jnp.float32),
    )(ret_u, bsc_u, inp3, it512)

    return out.reshape(input.shape)


# speedup vs baseline: 175.3994x; 1.2969x over previous
import jax
import jax.numpy as jnp
import numpy as np
from jax import lax
from jax.experimental import pallas as pl
from jax.experimental.pallas import tpu as pltpu
from jax.experimental.pallas import tpu_sc as plsc

NH = 4
NB = 64
C = 16
D = 64
CH = 144
L = 9216
NCH = L // CH
BLK = 128
NBLK = L // BLK


A1B = 1536
PAD = 128


def _a1_body(x_ref, w_ref, fw1_ref, fb1_ref, fw2_ref,
             fb2_ref, xe_ref, y_ref, mlp_ref):
    for j in range(L // A1B):
        col = lax.rem(lax.broadcasted_iota(jnp.int32, (64, A1B), 1)
                      + (j * A1B), 96)
        mask_first = (col != 0).astype(jnp.float32)
        mask_last = (col != 95).astype(jnp.float32)
        acc = jnp.zeros((A1B, 144), jnp.float32)
        for ky in range(3):
            for kx in range(3):
                sh = (ky - 1) * 96 + (kx - 1)
                Xt = x_ref[0, :, pl.ds(PAD + j * A1B + sh, A1B)]
                if kx == 0:
                    Xt = Xt * mask_first
                elif kx == 2:
                    Xt = Xt * mask_last
                acc = acc + lax.dot_general(Xt, w_ref[ky, kx],
                                            (((0,), (1,)), ((), ())))
        fc = acc[:, 80:144]
        xe_ref[0, pl.ds(j * A1B, A1B), :] = acc[:, 0:16]
        y_ref[0, pl.ds(j * A1B, A1B), :] = acc[:, 16:80]
        h1 = jax.nn.relu(
            lax.dot_general(fc, fw1_ref[...], (((1,), (1,)), ((), ())))
            + fb1_ref[...][None, :])
        mlp_ref[0, pl.ds(j * A1B, A1B), :] = \
            lax.dot_general(h1, fw2_ref[...], (((1,), (1,)), ((), ()))) \
            + fb2_ref[...][None, :]


def _a2_body(xe_ref, rot_ref, tinc_ref, texc_ref, slb_ref, g_ref, pos_ref,
             scr_ref):
    xe = xe_ref[0]
    ohs = []
    for h in range(NH):
        r = lax.dot_general(xe, rot_ref[...][:, h * NB:(h + 1) * NB],
                            (((1,), (0,)), ((), ())))
        m = jnp.max(r, axis=1, keepdims=True)
        io = lax.broadcasted_iota(jnp.int32, (L, NB), 1).astype(jnp.float32)
        code = jnp.min(jnp.where(r == m, io, 1e9), axis=1, keepdims=True)
        ohs.append((io == code).astype(jnp.float32))
    O = jnp.concatenate(ohs, axis=1)
    bsums = []
    for b in range(NBLK):
        osl = lax.slice(O, (b * BLK, 0), (b * BLK + BLK, NH * NB))
        incl = lax.dot_general(tinc_ref[...], osl, (((1,), (0,)), ((), ())))
        scr_ref[pl.ds(b * BLK, BLK), :] = incl
        bsums.append(jnp.sum(osl, axis=0, keepdims=True))
    bs = jnp.concatenate(bsums, axis=0)
    prev = lax.dot_general(texc_ref[...], bs, (((1,), (0,)), ((), ())))
    hist = jnp.sum(bs, axis=0, keepdims=True)
    excl = lax.dot_general(hist, slb_ref[...], (((1,), (0,)), ((), ())))
    for b in range(NBLK):
        osl = lax.slice(O, (b * BLK, 0), (b * BLK + BLK, NH * NB))
        val = scr_ref[pl.ds(b * BLK, BLK), :] - 1.0 \
            + prev[b:b + 1, :] + excl
        posb = lax.dot_general(osl * val, g_ref[...], (((1,), (0,)), ((), ())))
        pos_ref[0, pl.ds(b * BLK, BLK), :] = posb.astype(jnp.int32)


def _b_body(gpos_hbm, x_hbm, y_hbm, mlp_hbm, xs_hbm, ys_hbm, ms_hbm,
            idx_v, bx, by, bm, sem):
    wid = lax.axis_index("s") * 2 + lax.axis_index("c")
    bh = wid // 4
    q = wid % 4
    b = bh // NH
    t0 = q * (L // 4)
    nchunks = (L // 4) // BLK
    for j in range(nchunks):
        goff = bh * L + t0 + j * BLK
        soff = b * L + t0 + j * BLK
        pltpu.sync_copy(gpos_hbm.at[pl.ds(goff, BLK)], idx_v.at[j])
        pltpu.sync_copy(x_hbm.at[pl.ds(soff, BLK)], bx)
        pltpu.async_copy(bx, xs_hbm.at[idx_v.at[j]], sem).wait()
        pltpu.sync_copy(y_hbm.at[pl.ds(soff, BLK)], by)
        pltpu.async_copy(by, ys_hbm.at[idx_v.at[j]], sem).wait()
        pltpu.sync_copy(mlp_hbm.at[pl.ds(soff, BLK)], bm)
        pltpu.async_copy(bm, ms_hbm.at[idx_v.at[j]], sem).wait()


CG = 4
QB = CG * CH
KB = QB + 2 * CH


def _c_body(xc_ref, xp_ref, xn_ref, yc_ref, yp_ref, yn_ref, mc_ref, mp_ref,
            mn_ref, msk_ref, ret_ref, bsc_ref):
    xq = xc_ref[0, 0]
    xk = jnp.concatenate(
        [xp_ref[0, 0, QB - CH:, :], xq, xn_ref[0, 0, :CH, :]], axis=0)
    nrm = jnp.sqrt(jnp.sum(xk * xk, axis=1, keepdims=True))
    xmn = xk / jnp.maximum(nrm, 5e-05)
    mk = jnp.concatenate(
        [mp_ref[0, 0, QB - CH:, :], mc_ref[0, 0], mn_ref[0, 0, :CH, :]],
        axis=0)
    mk4 = jnp.concatenate([mk] * CG, axis=1)
    rawT = lax.dot_general(xmn, xq, (((1,), (1,)), ((), ()))) + mk4 \
        + msk_ref[...]
    mx = jnp.max(rawT, axis=0, keepdims=True)
    ex = jnp.exp(rawT - mx)
    s = jnp.sum(ex, axis=0, keepdims=True)
    bsc_ref[0, 0, 0] = jnp.log(s) + mx
    scT = ex * (1.0 / s)
    yk = jnp.concatenate(
        [yp_ref[0, 0, QB - CH:, :], yc_ref[0, 0], yn_ref[0, 0, :CH, :]],
        axis=0)
    ret_ref[0, 0] = lax.dot_general(scT, yk, (((0,), (0,)), ((), ())))


def _d_body(gpos_hbm, rets_hbm, bscs_hbm, retu_hbm, bscu_hbm,
            idx_v, buf, scv, obuf, sem):
    wid = lax.axis_index("s") * 2 + lax.axis_index("c")
    bh = wid // 4
    q = wid % 4
    t0 = q * (L // 4)
    nchunks = (L // 4) // BLK
    pltpu.sync_copy(bscs_hbm.at[pl.ds(bh * L, L)], scv)
    base = jnp.full((16,), bh * L, jnp.int32)
    for j in range(nchunks):
        goff = bh * L + t0 + j * BLK
        pltpu.sync_copy(gpos_hbm.at[pl.ds(goff, BLK)], idx_v.at[j])
        pltpu.async_copy(rets_hbm.at[idx_v.at[j]], buf, sem).wait()
        pltpu.sync_copy(buf, retu_hbm.at[pl.ds(goff, BLK)])
        for g in range(BLK // 16):
            iv = idx_v[j, pl.ds(g * 16, 16)] - base
            obuf[pl.ds(j * BLK + g * 16, 16)] = plsc.load_gather(scv, [iv])
    pltpu.sync_copy(obuf, bscu_hbm.at[pl.ds(bh * L + t0, L // 4)])


def _e_body(ret_ref, bsc_ref, inp_ref, it_ref, out_ref):
    bsc = bsc_ref[0]
    i4 = jnp.eye(NH, dtype=jnp.float32)
    bscT = lax.dot_general(bsc, i4, (((0,), (0,)), ((), ())))
    mx = jnp.max(bscT, axis=1, keepdims=True)
    e = jnp.exp(bscT - mx)
    p = e / jnp.sum(e, axis=1, keepdims=True)
    acc = jnp.zeros((512, D), jnp.float32)
    for h in range(NH):
        acc = acc + ret_ref[0, h] * p[:, h:h + 1]
    outc = lax.dot_general(acc, it_ref[...], (((0,), (0,)), ((), ())))
    out_ref[0] = outc + inp_ref[0]


def _sc_scatter(gpos, x_flat, y_flat, mlp_flat):
    mesh = plsc.VectorSubcoreMesh(core_axis_name="c", subcore_axis_name="s")
    M = 2 * NH * L
    nchunks = (L // 4) // BLK
    f = pl.kernel(
        _b_body,
        out_type=(
            jax.ShapeDtypeStruct((M, C), jnp.float32),
            jax.ShapeDtypeStruct((M, D), jnp.float32),
            jax.ShapeDtypeStruct((M, CH), jnp.float32),
        ),
        mesh=mesh,
        scratch_types=[
            pltpu.VMEM((nchunks, BLK), jnp.int32),
            pltpu.VMEM((BLK, C), jnp.float32),
            pltpu.VMEM((BLK, D), jnp.float32),
            pltpu.VMEM((BLK, CH), jnp.float32),
            pltpu.SemaphoreType.DMA,
        ],
        compiler_params=pltpu.CompilerParams(use_tc_tiling_on_sc=False, needs_layout_passes=False),
    )
    return f(gpos, x_flat, y_flat, mlp_flat)


def _sc_gather(gpos, ret_s, bsc_s):
    mesh = plsc.VectorSubcoreMesh(core_axis_name="c", subcore_axis_name="s")
    M = 2 * NH * L
    nchunks = (L // 4) // BLK
    f = pl.kernel(
        _d_body,
        out_type=(
            jax.ShapeDtypeStruct((M, D), jnp.float32),
            jax.ShapeDtypeStruct((M,), jnp.float32),
        ),
        mesh=mesh,
        scratch_types=[
            pltpu.VMEM((nchunks, BLK), jnp.int32),
            pltpu.VMEM((BLK, D), jnp.float32),
            pltpu.VMEM((L,), jnp.float32),
            pltpu.VMEM((L // 4,), jnp.float32),
            pltpu.SemaphoreType.DMA,
        ],
        compiler_params=pltpu.CompilerParams(use_tc_tiling_on_sc=False, needs_layout_passes=False),
    )
    return f(gpos, ret_s, bsc_s)


def kernel(input, w_match, w_assembly, w_assembly_fc, fc_w1, fc_b1, fc_w2,
           fc_b2, rotations):
    N = input.shape[0]
    Wc = jnp.concatenate([w_match, w_assembly, w_assembly_fc], axis=0)
    Wc = Wc.transpose(2, 3, 0, 1)
    rot2 = rotations.reshape(C, NH * NB)
    inp3 = input.reshape(N, 64, L)
    inp_pad = jnp.pad(inp3, ((0, 0), (0, 0), (PAD, PAD)))

    xe, y, mlp = pl.pallas_call(
        _a1_body,
        grid=(N,),
        in_specs=[
            pl.BlockSpec((1, 64, L + 2 * PAD), lambda n: (n, 0, 0)),
            pl.BlockSpec((3, 3, 144, 64), lambda n: (0, 0, 0, 0)),
            pl.BlockSpec((144, 64), lambda n: (0, 0)),
            pl.BlockSpec((144,), lambda n: (0,)),
            pl.BlockSpec((144, 144), lambda n: (0, 0)),
            pl.BlockSpec((144,), lambda n: (0,)),
        ],
        out_specs=[
            pl.BlockSpec((1, L, C), lambda n: (n, 0, 0)),
            pl.BlockSpec((1, L, D), lambda n: (n, 0, 0)),
            pl.BlockSpec((1, L, CH), lambda n: (n, 0, 0)),
        ],
        out_shape=[
            jax.ShapeDtypeStruct((N, L, C), jnp.float32),
            jax.ShapeDtypeStruct((N, L, D), jnp.float32),
            jax.ShapeDtypeStruct((N, L, CH), jnp.float32),
        ],
    )(inp_pad, Wc, fc_w1, fc_b1, fc_w2, fc_b2)

    tinc = jnp.asarray(np.tril(np.ones((BLK, BLK))), jnp.float32)
    texc = jnp.asarray(np.tril(np.ones((NBLK, NBLK)), k=-1), jnp.float32)
    dd = np.arange(NB)
    slb_small = (dd[:, None] < dd[None, :]).astype(np.float32)
    slb = jnp.asarray(np.kron(np.eye(NH), slb_small), jnp.float32)
    gmat = jnp.asarray(np.kron(np.eye(NH), np.ones((NB, 1))), jnp.float32)

    pos = pl.pallas_call(
        _a2_body,
        grid=(N,),
        in_specs=[
            pl.BlockSpec((1, L, C), lambda n: (n, 0, 0)),
            pl.BlockSpec((C, NH * NB), lambda n: (0, 0)),
            pl.BlockSpec((BLK, BLK), lambda n: (0, 0)),
            pl.BlockSpec((NBLK, NBLK), lambda n: (0, 0)),
            pl.BlockSpec((NH * NB, NH * NB), lambda n: (0, 0)),
            pl.BlockSpec((NH * NB, NH), lambda n: (0, 0)),
        ],
        out_specs=pl.BlockSpec((1, L, NH), lambda n: (n, 0, 0)),
        out_shape=jax.ShapeDtypeStruct((N, L, NH), jnp.int32),
        scratch_shapes=[pltpu.VMEM((L, NH * NB), jnp.float32)],
    )(xe, rot2, tinc, texc, slb, gmat)

    offs = (jnp.arange(N, dtype=jnp.int32)[:, None, None] * NH
            + jnp.arange(NH, dtype=jnp.int32)[None, :, None]) * L
    gpos = (pos.transpose(0, 2, 1) + offs).reshape(-1)

    xs, ys, ms = _sc_scatter(gpos, xe.reshape(N * L, C), y.reshape(N * L, D),
                             mlp.reshape(N * L, CH))

    xs4 = xs.reshape(N, NH, L, C)
    ys4 = ys.reshape(N, NH, L, D)
    ms4 = ms.reshape(N, NH, L, CH)

    NG = NCH // CG

    def ctr(n, h, k):
        return (n, h, k, 0)

    def prv(n, h, k):
        return (n, h, (k - 1) % NG, 0)

    def nxt(n, h, k):
        return (n, h, (k + 1) % NG, 0)

    def mkspec(dim, imap):
        return pl.BlockSpec((1, 1, QB, dim), imap)

    cj = np.arange(KB) // CH
    ci = np.arange(QB) // CH + 1
    maskadd = jnp.asarray(
        np.where(np.abs(cj[:, None] - ci[None, :]) <= 1, 0.0, -1e30),
        jnp.float32)

    ret_s, bsc_s = pl.pallas_call(
        _c_body,
        grid=(N, NH, NG),
        in_specs=[mkspec(C, ctr), mkspec(C, prv), mkspec(C, nxt),
                  mkspec(D, ctr), mkspec(D, prv), mkspec(D, nxt),
                  mkspec(CH, ctr), mkspec(CH, prv), mkspec(CH, nxt),
                  pl.BlockSpec((KB, QB), lambda n, h, k: (0, 0))],
        out_specs=[
            pl.BlockSpec((1, 1, QB, D),
                         lambda n, h, k: (n, h, k, 0)),
            pl.BlockSpec((1, 1, 1, 1, QB),
                         lambda n, h, k: (n, h, k, 0, 0)),
        ],
        out_shape=[
            jax.ShapeDtypeStruct((N, NH, L, D), jnp.float32),
            jax.ShapeDtypeStruct((N, NH, NG, 1, QB), jnp.float32),
        ],
    )(xs4, xs4, xs4, ys4, ys4, ys4, ms4, ms4, ms4, maskadd)

    ret_u, bsc_u = _sc_gather(gpos, ret_s.reshape(N * NH * L, D),
                              bsc_s.reshape(N * NH * L))

    ret_u = ret_u.reshape(N, NH, L, D)
    bsc_u = bsc_u.reshape(N, NH, L)
    it512 = jnp.asarray(np.eye(512), jnp.float32)
    inp3 = input.reshape(N, 64, L)

    TB = 512
    out = pl.pallas_call(
        _e_body,
        grid=(N, L // TB),
        in_specs=[
            pl.BlockSpec((1, NH, TB, D), lambda n, t: (n, 0, t, 0)),
            pl.BlockSpec((1, NH, TB), lambda n, t: (n, 0, t)),
            pl.BlockSpec((1, 64, TB), lambda n, t: (n, 0, t)),
            pl.BlockSpec((TB, TB), lambda n, t: (0, 0)),
        ],
        out_specs=pl.BlockSpec((1, 64, TB), lambda n, t: (n, 0, t)),
        out_shape=jax.ShapeDtypeStruct((N, 64, L), jnp.float32),
    )(ret_u, bsc_u, inp3, it512)

    return out.reshape(input.shape)
